# baseline (speedup 1.0000x reference)
import functools

import jax
import jax.numpy as jnp
from jax import lax
from jax.experimental import pallas as pl
from jax.experimental.pallas import tpu as pltpu
from jax.experimental.pallas import tpu_sc as plsc

N = 10000
D = 128
H = 128
DIM = 3

NC = 2
NS = 16
NW = NC * NS
L = 16

NPAD = 10240
STRIPE = NPAD // NS
TRASH = N

CH = 128
NSLICE = 2
MCH = 40
EB = MCH * CH * NSLICE
EPAD = NW * EB
ESL = EPAD // NSLICE

_MID_BLK = 2048
_PRE_BLK = 640

_sc_mesh = plsc.VectorSubcoreMesh(core_axis_name="c", subcore_axis_name="s")


def _elu(v):
    return jnp.where(v > 0, v, jnp.exp(v) - 1.0)


def _tc_pre_body(xc_ref, wsrc_ref, wdst_ref, gsrc_ref, gdst_ref):
    xc = xc_ref[...]
    gsrc_ref[...] = jnp.dot(xc, wsrc_ref[...], preferred_element_type=jnp.float32)
    gdst_ref[...] = jnp.dot(xc, wdst_ref[...], preferred_element_type=jnp.float32)


def _tc_pre(xc, wsrc, wdst):
    grid = NPAD // _PRE_BLK
    return pl.pallas_call(
        _tc_pre_body,
        grid=(grid,),
        in_specs=[
            pl.BlockSpec((_PRE_BLK, 256), lambda i: (i, 0)),
            pl.BlockSpec((256, H), lambda i: (0, 0)),
            pl.BlockSpec((256, H), lambda i: (0, 0)),
        ],
        out_specs=[
            pl.BlockSpec((_PRE_BLK, H), lambda i: (i, 0)),
            pl.BlockSpec((_PRE_BLK, H), lambda i: (i, 0)),
        ],
        out_shape=[
            jax.ShapeDtypeStruct((NPAD, H), jnp.float32),
            jax.ShapeDtypeStruct((NPAD, H), jnp.float32),
        ],
    )(xc, wsrc, wdst)


def _make_gather(m):
    nedge = NW * m * CH
    ngrp = m // 4

    @functools.partial(
        pl.kernel,
        out_type=jax.ShapeDtypeStruct((nedge, H), jnp.float32),
        mesh=_sc_mesh,
        scratch_types=(
            [pltpu.VMEM((2, CH), jnp.int32) for _ in range(4)]
            + [pltpu.VMEM((CH, H), jnp.float32) for _ in range(6)]
            + [pltpu.SemaphoreType.DMA for _ in range(10)]
        ),
    )
    def gather(gsrc_hbm, gdst_hbm, eidx_hbm, out_hbm,
               i0, i1, i2, i3, a0, a1, b0, b1, c0, c1,
               si0, si1, si2, si3, sa0, sa1, sb0, sb1, sc0, sc1):
        IDX = (i0, i1, i2, i3)
        SI = (si0, si1, si2, si3)
        A = (a0, a1)
        B = (b0, b1)
        C = (c0, c1)
        SA = (sa0, sa1)
        SB = (sb0, sb1)
        SC = (sc0, sc1)

        wid = lax.axis_index("s") * NC + lax.axis_index("c")
        base = wid * (m * CH)
        cbase = wid * m

        def idx_start(k, s):
            pltpu.async_copy(eidx_hbm.at[cbase + k], IDX[s], SI[s])

        def idx_wait(k, s):
            pltpu.make_async_copy(eidx_hbm.at[cbase + k], IDX[s], SI[s]).wait()

        def gat_start(q, s):
            pltpu.async_copy(gsrc_hbm.at[IDX[s].at[0]], A[q], SA[q])
            pltpu.async_copy(gdst_hbm.at[IDX[s].at[1]], B[q], SB[q])

        def gat_wait(q, s):
            pltpu.make_async_copy(gsrc_hbm.at[IDX[s].at[0]], A[q], SA[q]).wait()
            pltpu.make_async_copy(gdst_hbm.at[IDX[s].at[1]], B[q], SB[q]).wait()

        def add(q):
            @plsc.parallel_loop(0, CH, 1, unroll=2)
            def _(r):
                for j in range(H // L):
                    sl = pl.ds(j * L, L)
                    C[q][r, sl] = A[q][r, sl] + B[q][r, sl]

        def wb_start(k, q):
            pltpu.async_copy(C[q], out_hbm.at[pl.ds(base + k * CH, CH)], SC[q])

        def wb_wait(k, q):
            pltpu.make_async_copy(
                C[q], out_hbm.at[pl.ds(base + k * CH, CH)], SC[q]).wait()

        pltpu.sync_copy(eidx_hbm.at[cbase + 0], IDX[0])
        pltpu.sync_copy(eidx_hbm.at[cbase + 1], IDX[1])
        gat_start(0, 0)
        gat_start(1, 1)
        idx_start(2, 2)
        idx_start(3, 3)

        def group(g, _):
            for p in range(4):
                q = p % 2
                k = 4 * g + p
                gat_wait(q, p)
                if p >= 2:
                    wb_wait(k - 2, q)
                else:
                    @pl.when(g > 0)
                    def _():
                        wb_wait(k - 2, q)
                add(q)
                wb_start(k, q)

                @pl.when(g < ngrp - 1)
                def _():
                    idx_start(k + 4, p)

                sn = (p + 2) % 4
                if p < 2:
                    idx_wait(k + 2, sn)
                    gat_start(q, sn)
                else:
                    @pl.when(g < ngrp - 1)
                    def _():
                        idx_wait(k + 2, sn)
                        gat_start(q, sn)
            return 0

        lax.fori_loop(0, ngrp, group, 0)

        wb_wait(m - 2, 0)
        wb_wait(m - 1, 1)

    return gather


_sc_gather = _make_gather(MCH)


def _tc_mid_body(h0_ref, be0_ref, we2_ref, be2_ref, out_ref):
    h0 = _elu(h0_ref[...] + be0_ref[...])
    out_ref[...] = _elu(
        jnp.dot(h0, we2_ref[...], preferred_element_type=jnp.float32)
        + be2_ref[...])


def _tc_mid(h0pre, be0, We2, be2):
    rows = h0pre.shape[0]
    return pl.pallas_call(
        _tc_mid_body,
        grid=(rows // _MID_BLK,),
        in_specs=[
            pl.BlockSpec((_MID_BLK, H), lambda i: (i, 0)),
            pl.BlockSpec((1, H), lambda i: (0, 0)),
            pl.BlockSpec((H, H), lambda i: (0, 0)),
            pl.BlockSpec((1, H), lambda i: (0, 0)),
        ],
        out_specs=pl.BlockSpec((_MID_BLK, H), lambda i: (i, 0)),
        out_shape=jax.ShapeDtypeStruct((rows, H), jnp.float32),
    )(h0pre, be0, We2, be2)


def _make_scatter(m):
    @functools.partial(
        pl.kernel,
        out_type=[
            jax.ShapeDtypeStruct((NC * NPAD, H), jnp.float32),
            jax.ShapeDtypeStruct((NC * NPAD,), jnp.float32),
        ],
        mesh=_sc_mesh,
        scratch_types=(
            [pltpu.VMEM((CH,), jnp.int32) for _ in range(2)]
            + [pltpu.VMEM((CH, H), jnp.float32) for _ in range(2)]
            + [
                pltpu.VMEM((CH,), jnp.float32),
                pltpu.VMEM((STRIPE,), jnp.float32),
                pltpu.VMEM_SHARED((NPAD, H), jnp.float32),
                pltpu.VMEM_SHARED((NPAD,), jnp.float32),
            ]
            + [pltpu.SemaphoreType.DMA for _ in range(8)]
        ),
    )
    def scatter(h1_hbm, srcs_hbm, s_hbm, deg_hbm,
                x0, x1, v0, v1, ones_v, dbuf, acc, dacc,
                sx0, sx1, sv0, sv1, ss0, ss1, so0, so1):
        IDX = (x0, x1)
        V = (v0, v1)
        SX = (sx0, sx1)
        SV = (sv0, sv1)
        SS = (ss0, ss1)
        SO = (so0, so1)

        cid = lax.axis_index("c")
        sid = lax.axis_index("s")
        wid = sid * NC + cid
        sb = sid * STRIPE
        base = wid * (m * CH)

        def zero_vrow(r, _):
            for j in range(H // L):
                v0[r, pl.ds(j * L, L)] = jnp.zeros((L,), jnp.float32)
            return 0

        lax.fori_loop(0, CH, zero_vrow, 0)

        def zero_d(i, _):
            dbuf[pl.ds(i * L, L)] = jnp.zeros((L,), jnp.float32)
            return 0

        lax.fori_loop(0, STRIPE // L, zero_d, 0)

        for j in range(CH // L):
            ones_v[pl.ds(j * L, L)] = jnp.full((L,), 1.0, jnp.float32)

        for t in range(STRIPE // CH):
            pltpu.sync_copy(v0, acc.at[pl.ds(sb + t * CH, CH)])
        pltpu.sync_copy(dbuf, dacc.at[pl.ds(sb, STRIPE)])
        plsc.subcore_barrier()

        def ld_start(k, s):
            off = base + k * CH
            pltpu.async_copy(srcs_hbm.at[pl.ds(off, CH)], IDX[s], SX[s])
            pltpu.async_copy(h1_hbm.at[pl.ds(off, CH)], V[s], SV[s])

        def ld_wait(k, s):
            off = base + k * CH
            pltpu.make_async_copy(
                srcs_hbm.at[pl.ds(off, CH)], IDX[s], SX[s]).wait()
            pltpu.make_async_copy(
                h1_hbm.at[pl.ds(off, CH)], V[s], SV[s]).wait()

        def sc_start(s):
            pltpu.async_copy(V[s], acc.at[IDX[s]], SS[s], add=True)
            pltpu.async_copy(ones_v, dacc.at[IDX[s]], SO[s], add=True)

        def sc_wait(s):
            pltpu.make_async_copy(V[s], acc.at[IDX[s]], SS[s]).wait()
            pltpu.make_async_copy(ones_v, dacc.at[IDX[s]], SO[s]).wait()

        ld_start(0, 0)
        ld_start(1, 1)

        def pair(j, _):
            k = 2 * j
            ld_wait(k, 0)
            sc_start(0)
            ld_wait(k + 1, 1)
            sc_start(1)

            @pl.when(j < m // 2 - 1)
            def _():
                sc_wait(0)
                ld_start(k + 2, 0)
                sc_wait(1)
                ld_start(k + 3, 1)

            return 0

        lax.fori_loop(0, m // 2, pair, 0)
        sc_wait(0)
        sc_wait(1)
        plsc.subcore_barrier()

        for t in range(STRIPE // CH):
            pltpu.sync_copy(acc.at[pl.ds(sb + t * CH, CH)], v0)
            pltpu.sync_copy(v0, s_hbm.at[pl.ds(cid * NPAD + sb + t * CH, CH)])
        pltpu.sync_copy(dacc.at[pl.ds(sb, STRIPE)], dbuf)
        pltpu.sync_copy(dbuf, deg_hbm.at[pl.ds(cid * NPAD + sb, STRIPE)])

    return scatter


_sc_scatter = _make_scatter(MCH)


NPART = NSLICE * NC


def _tc_post_body(*refs):
    s_refs = refs[:NPART]
    d_refs = refs[NPART:2 * NPART]
    (x_ref, we3_ref, be3_ref, wn0a_ref, wn0b_ref, bn0_ref,
     wn2_ref, bn2_ref, wn3_ref, bn3_ref,
     wd0_ref, bd0_ref, wd2_ref, bd2_ref, wd3_ref, bd3_ref,
     out_ref) = refs[2 * NPART:]
    f32 = jnp.float32
    ssum = s_refs[0][...]
    for r in s_refs[1:]:
        ssum = ssum + r[...]
    deg = d_refs[0][...]
    for r in d_refs[1:]:
        deg = deg + r[...]
    es = (jnp.dot(ssum, we3_ref[...], preferred_element_type=f32)
          + deg * be3_ref[...])
    h = _elu(jnp.dot(x_ref[...], wn0a_ref[...], preferred_element_type=f32)
             + jnp.dot(es, wn0b_ref[...], preferred_element_type=f32)
             + bn0_ref[...])
    h = _elu(jnp.dot(h, wn2_ref[...], preferred_element_type=f32) + bn2_ref[...])
    na = jnp.dot(h, wn3_ref[...], preferred_element_type=f32) + bn3_ref[...]
    h = _elu(jnp.dot(na, wd0_ref[...], preferred_element_type=f32) + bd0_ref[...])
    h = _elu(jnp.dot(h, wd2_ref[...], preferred_element_type=f32) + bd2_ref[...])
    out_ref[...] = jnp.dot(h, wd3_ref[...], preferred_element_type=f32) + bd3_ref[...]


def _tc_post(s_parts, d_parts, xpad, We3, be3, Wn0a, Wn0b, bn0,
             Wn2, bn2, Wn3, bn3, Wd0, bd0, Wd2, bd2, Wd3, bd3):
    grid = NPAD // _PRE_BLK
    row_blk = lambda i: (i, 0)
    full = lambda i: (0, 0)
    wspec = pl.BlockSpec((H, H), full)
    bspec = pl.BlockSpec((1, H), full)
    return pl.pallas_call(
        _tc_post_body,
        grid=(grid,),
        in_specs=(
            [pl.BlockSpec((_PRE_BLK, H), row_blk)] * NPART
            + [pl.BlockSpec((_PRE_BLK, 1), row_blk)] * NPART
            + [pl.BlockSpec((_PRE_BLK, H), row_blk),
               wspec, bspec, wspec, wspec, bspec,
               wspec, bspec, wspec, bspec,
               wspec, bspec, wspec, bspec, wspec, bspec]
        ),
        out_specs=pl.BlockSpec((_PRE_BLK, H), row_blk),
        out_shape=jax.ShapeDtypeStruct((NPAD, H), jnp.float32),
    )(*s_parts, *d_parts, xpad, We3, be3, Wn0a, Wn0b, bn0,
      Wn2, bn2, Wn3, bn3, Wd0, bd0, Wd2, bd2, Wd3, bd3)


def kernel(x, edge_index, pos, We0, be0, We2, be2, We3, be3,
           Wn0, bn0, Wn2, bn2, Wn3, bn3, Wd0, bd0, Wd2, bd2, Wd3, bd3):
    f32 = jnp.float32
    src = edge_index[0]
    dst = edge_index[1]
    E = src.shape[0]

    Wg = We0[:DIM]
    Wi = We0[DIM:DIM + D]
    Wj = We0[DIM + D:]
    zfill = jnp.zeros((256 - D - DIM, H), f32)
    wsrc = jnp.concatenate([Wi, -Wg, zfill], axis=0)
    wdst = jnp.concatenate([Wj, Wg, zfill], axis=0)

    xc = jnp.concatenate([x, pos], axis=1)
    xc = jnp.pad(xc, ((0, NPAD - N), (0, 256 - (D + DIM))))

    gsrc, gdst = _tc_pre(xc, wsrc, wdst)

    epad = EPAD - E
    srcg = jnp.pad(src, (0, epad))
    dstg = jnp.pad(dst, (0, epad))
    srcs = jnp.pad(src, (0, epad), constant_values=TRASH)
    eidx = jnp.stack(
        [srcg.reshape(-1, CH), dstg.reshape(-1, CH)], axis=1)

    csl = ESL // CH
    order = list(reversed(range(NSLICE)))
    h0pre_d = {}
    h1_d = {}
    tok = jnp.zeros((1,), jnp.int32)
    for s in order:
        eidx_s = eidx[s * csl:(s + 1) * csl] + tok
        h0pre_s = _sc_gather(gsrc, gdst, eidx_s)
        h1_s = _tc_mid(h0pre_s, be0.reshape(1, H), We2, be2.reshape(1, H))
        tok = (h1_s[0, :1] * 0.0).astype(jnp.int32)
        h0pre_d[s] = h0pre_s
        h1_d[s] = h1_s

    s_parts = []
    d_parts = []
    for i, s in enumerate(order):
        srcs_s = srcs[s * ESL:(s + 1) * ESL]
        if i + 1 < len(order):
            nxt = order[i + 1]
            srcs_s = srcs_s + (h0pre_d[nxt][0, :1] * 0.0).astype(jnp.int32)
        s_all, deg_all = _sc_scatter(h1_d[s], srcs_s)
        s_parts += [s_all[:NPAD], s_all[NPAD:]]
        d_parts += [deg_all[:NPAD].reshape(NPAD, 1),
                    deg_all[NPAD:].reshape(NPAD, 1)]

    xpad = jnp.pad(x, ((0, NPAD - N), (0, 0)))

    out_pad = _tc_post(s_parts, d_parts, xpad,
                       We3, be3.reshape(1, H), Wn0[:D], Wn0[D:],
                       bn0.reshape(1, H), Wn2, bn2.reshape(1, H),
                       Wn3, bn3.reshape(1, H), Wd0, bd0.reshape(1, H),
                       Wd2, bd2.reshape(1, H), Wd3, bd3.reshape(1, H))
    return out_pad[:N]


# speedup vs baseline: 1.6635x; 1.6635x over previous
import functools

import jax
import jax.numpy as jnp
from jax import lax
from jax.experimental import pallas as pl
from jax.experimental.pallas import tpu as pltpu
from jax.experimental.pallas import tpu_sc as plsc

N = 10000
D = 128
H = 128
DIM = 3

NC = 2
NS = 16
NW = NC * NS
L = 16

NPAD = 10240
STRIPE = NPAD // NS
TRASH = N

CH = 128
NSLICE = 2
MCH = 40
EB = MCH * CH * NSLICE
EPAD = NW * EB
ESL = EPAD // NSLICE

_MID_BLK = 2048
_PRE_BLK = 640

_sc_mesh = plsc.VectorSubcoreMesh(core_axis_name="c", subcore_axis_name="s")


def _elu(v):
    return jnp.where(v > 0, v, jnp.exp(v) - 1.0)


def _tc_pre_body(xc_ref, wsrc_ref, wdst_ref, gsrc_ref, gdst_ref):
    xc = xc_ref[...]
    gsrc_ref[...] = jnp.dot(xc, wsrc_ref[...], preferred_element_type=jnp.float32)
    gdst_ref[...] = jnp.dot(xc, wdst_ref[...], preferred_element_type=jnp.float32)


def _tc_pre(xc, wsrc, wdst):
    grid = NPAD // _PRE_BLK
    return pl.pallas_call(
        _tc_pre_body,
        grid=(grid,),
        in_specs=[
            pl.BlockSpec((_PRE_BLK, 256), lambda i: (i, 0)),
            pl.BlockSpec((256, H), lambda i: (0, 0)),
            pl.BlockSpec((256, H), lambda i: (0, 0)),
        ],
        out_specs=[
            pl.BlockSpec((_PRE_BLK, H), lambda i: (i, 0)),
            pl.BlockSpec((_PRE_BLK, H), lambda i: (i, 0)),
        ],
        out_shape=[
            jax.ShapeDtypeStruct((NPAD, H), jnp.float32),
            jax.ShapeDtypeStruct((NPAD, H), jnp.float32),
        ],
    )(xc, wsrc, wdst)


def _make_gather(m):
    nedge = NW * m * CH
    ngrp = m // 4

    @functools.partial(
        pl.kernel,
        out_type=jax.ShapeDtypeStruct((nedge, H), jnp.float32),
        mesh=_sc_mesh,
        scratch_types=(
            [pltpu.VMEM((2, CH), jnp.int32) for _ in range(4)]
            + [pltpu.VMEM((CH, H), jnp.float32) for _ in range(6)]
            + [pltpu.SemaphoreType.DMA for _ in range(10)]
        ),
    )
    def gather(gsrc_hbm, gdst_hbm, eidx_hbm, out_hbm,
               i0, i1, i2, i3, a0, a1, b0, b1, c0, c1,
               si0, si1, si2, si3, sa0, sa1, sb0, sb1, sc0, sc1):
        IDX = (i0, i1, i2, i3)
        SI = (si0, si1, si2, si3)
        A = (a0, a1)
        B = (b0, b1)
        C = (c0, c1)
        SA = (sa0, sa1)
        SB = (sb0, sb1)
        SC = (sc0, sc1)

        wid = lax.axis_index("s") * NC + lax.axis_index("c")
        base = wid * (m * CH)
        cbase = wid * m

        def idx_start(k, s):
            pltpu.async_copy(eidx_hbm.at[cbase + k], IDX[s], SI[s])

        def idx_wait(k, s):
            pltpu.make_async_copy(eidx_hbm.at[cbase + k], IDX[s], SI[s]).wait()

        def gat_start(q, s):
            pltpu.async_copy(gsrc_hbm.at[IDX[s].at[0]], A[q], SA[q])
            pltpu.async_copy(gdst_hbm.at[IDX[s].at[1]], B[q], SB[q])

        def gat_wait(q, s):
            pltpu.make_async_copy(gsrc_hbm.at[IDX[s].at[0]], A[q], SA[q]).wait()
            pltpu.make_async_copy(gdst_hbm.at[IDX[s].at[1]], B[q], SB[q]).wait()

        def add(q):
            @plsc.parallel_loop(0, CH, 1, unroll=2)
            def _(r):
                for j in range(H // L):
                    sl = pl.ds(j * L, L)
                    C[q][r, sl] = A[q][r, sl] + B[q][r, sl]

        def wb_start(k, q):
            pltpu.async_copy(C[q], out_hbm.at[pl.ds(base + k * CH, CH)], SC[q])

        def wb_wait(k, q):
            pltpu.make_async_copy(
                C[q], out_hbm.at[pl.ds(base + k * CH, CH)], SC[q]).wait()

        pltpu.sync_copy(eidx_hbm.at[cbase + 0], IDX[0])
        pltpu.sync_copy(eidx_hbm.at[cbase + 1], IDX[1])
        gat_start(0, 0)
        gat_start(1, 1)
        idx_start(2, 2)
        idx_start(3, 3)

        def group(g, _):
            for p in range(4):
                q = p % 2
                k = 4 * g + p
                gat_wait(q, p)
                if p >= 2:
                    wb_wait(k - 2, q)
                else:
                    @pl.when(g > 0)
                    def _():
                        wb_wait(k - 2, q)
                add(q)
                wb_start(k, q)

                @pl.when(g < ngrp - 1)
                def _():
                    idx_start(k + 4, p)

                sn = (p + 2) % 4
                if p < 2:
                    idx_wait(k + 2, sn)
                    gat_start(q, sn)
                else:
                    @pl.when(g < ngrp - 1)
                    def _():
                        idx_wait(k + 2, sn)
                        gat_start(q, sn)
            return 0

        lax.fori_loop(0, ngrp, group, 0)

        wb_wait(m - 2, 0)
        wb_wait(m - 1, 1)

    return gather


_sc_gather = _make_gather(MCH)


def _tc_mid_body(h0_ref, be0_ref, we2_ref, be2_ref, out_ref):
    h0 = _elu(h0_ref[...] + be0_ref[...])
    out_ref[...] = _elu(
        jnp.dot(h0, we2_ref[...], preferred_element_type=jnp.float32)
        + be2_ref[...])


def _tc_mid(h0pre, be0, We2, be2):
    rows = h0pre.shape[0]
    return pl.pallas_call(
        _tc_mid_body,
        grid=(rows // _MID_BLK,),
        in_specs=[
            pl.BlockSpec((_MID_BLK, H), lambda i: (i, 0)),
            pl.BlockSpec((1, H), lambda i: (0, 0)),
            pl.BlockSpec((H, H), lambda i: (0, 0)),
            pl.BlockSpec((1, H), lambda i: (0, 0)),
        ],
        out_specs=pl.BlockSpec((_MID_BLK, H), lambda i: (i, 0)),
        out_shape=jax.ShapeDtypeStruct((rows, H), jnp.float32),
    )(h0pre, be0, We2, be2)


def _make_scatter(m):
    @functools.partial(
        pl.kernel,
        out_type=[
            jax.ShapeDtypeStruct((NC * NPAD, H), jnp.float32),
            jax.ShapeDtypeStruct((NC * NPAD,), jnp.float32),
        ],
        mesh=_sc_mesh,
        scratch_types=(
            [pltpu.VMEM((CH,), jnp.int32) for _ in range(2)]
            + [pltpu.VMEM((CH, H), jnp.float32) for _ in range(2)]
            + [
                pltpu.VMEM((CH,), jnp.float32),
                pltpu.VMEM((STRIPE,), jnp.float32),
                pltpu.VMEM_SHARED((NPAD, H), jnp.float32),
                pltpu.VMEM_SHARED((NPAD,), jnp.float32),
            ]
            + [pltpu.SemaphoreType.DMA for _ in range(8)]
        ),
    )
    def scatter(h1_hbm, srcs_hbm, s_hbm, deg_hbm,
                x0, x1, v0, v1, ones_v, dbuf, acc, dacc,
                sx0, sx1, sv0, sv1, ss0, ss1, so0, so1):
        IDX = (x0, x1)
        V = (v0, v1)
        SX = (sx0, sx1)
        SV = (sv0, sv1)
        SS = (ss0, ss1)
        SO = (so0, so1)

        cid = lax.axis_index("c")
        sid = lax.axis_index("s")
        wid = sid * NC + cid
        sb = sid * STRIPE
        base = wid * (m * CH)

        def zero_vrow(r, _):
            for j in range(H // L):
                v0[r, pl.ds(j * L, L)] = jnp.zeros((L,), jnp.float32)
            return 0

        lax.fori_loop(0, CH, zero_vrow, 0)

        def zero_d(i, _):
            dbuf[pl.ds(i * L, L)] = jnp.zeros((L,), jnp.float32)
            return 0

        lax.fori_loop(0, STRIPE // L, zero_d, 0)

        for j in range(CH // L):
            ones_v[pl.ds(j * L, L)] = jnp.full((L,), 1.0, jnp.float32)

        for t in range(STRIPE // CH):
            pltpu.sync_copy(v0, acc.at[pl.ds(sb + t * CH, CH)])
        pltpu.sync_copy(dbuf, dacc.at[pl.ds(sb, STRIPE)])
        plsc.subcore_barrier()

        def ld_start(k, s):
            off = base + k * CH
            pltpu.async_copy(srcs_hbm.at[pl.ds(off, CH)], IDX[s], SX[s])
            pltpu.async_copy(h1_hbm.at[pl.ds(off, CH)], V[s], SV[s])

        def ld_wait(k, s):
            off = base + k * CH
            pltpu.make_async_copy(
                srcs_hbm.at[pl.ds(off, CH)], IDX[s], SX[s]).wait()
            pltpu.make_async_copy(
                h1_hbm.at[pl.ds(off, CH)], V[s], SV[s]).wait()

        def sc_start(s):
            pltpu.async_copy(V[s], acc.at[IDX[s]], SS[s], add=True)
            pltpu.async_copy(ones_v, dacc.at[IDX[s]], SO[s], add=True)

        def sc_wait(s):
            pltpu.make_async_copy(V[s], acc.at[IDX[s]], SS[s]).wait()
            pltpu.make_async_copy(ones_v, dacc.at[IDX[s]], SO[s]).wait()

        ld_start(0, 0)
        ld_start(1, 1)

        def pair(j, _):
            k = 2 * j
            ld_wait(k, 0)
            sc_start(0)
            ld_wait(k + 1, 1)
            sc_start(1)

            @pl.when(j < m // 2 - 1)
            def _():
                sc_wait(0)
                ld_start(k + 2, 0)
                sc_wait(1)
                ld_start(k + 3, 1)

            return 0

        lax.fori_loop(0, m // 2, pair, 0)
        sc_wait(0)
        sc_wait(1)
        plsc.subcore_barrier()

        for t in range(STRIPE // CH):
            pltpu.sync_copy(acc.at[pl.ds(sb + t * CH, CH)], v0)
            pltpu.sync_copy(v0, s_hbm.at[pl.ds(cid * NPAD + sb + t * CH, CH)])
        pltpu.sync_copy(dacc.at[pl.ds(sb, STRIPE)], dbuf)
        pltpu.sync_copy(dbuf, deg_hbm.at[pl.ds(cid * NPAD + sb, STRIPE)])

    return scatter


_sc_scatter = _make_scatter(MCH)


NPART = NSLICE * NC


def _tc_post_body(*refs):
    s_refs = refs[:NPART]
    d_refs = refs[NPART:2 * NPART]
    (x_ref, we3_ref, be3_ref, wn0a_ref, wn0b_ref, bn0_ref,
     wn2_ref, bn2_ref, wn3_ref, bn3_ref,
     wd0_ref, bd0_ref, wd2_ref, bd2_ref, wd3_ref, bd3_ref,
     out_ref) = refs[2 * NPART:]
    f32 = jnp.float32
    ssum = s_refs[0][...][0]
    for r in s_refs[1:]:
        ssum = ssum + r[...][0]
    deg = d_refs[0][...][0]
    for r in d_refs[1:]:
        deg = deg + r[...][0]
    es = (jnp.dot(ssum, we3_ref[...], preferred_element_type=f32)
          + deg * be3_ref[...])
    h = _elu(jnp.dot(x_ref[...], wn0a_ref[...], preferred_element_type=f32)
             + jnp.dot(es, wn0b_ref[...], preferred_element_type=f32)
             + bn0_ref[...])
    h = _elu(jnp.dot(h, wn2_ref[...], preferred_element_type=f32) + bn2_ref[...])
    na = jnp.dot(h, wn3_ref[...], preferred_element_type=f32) + bn3_ref[...]
    h = _elu(jnp.dot(na, wd0_ref[...], preferred_element_type=f32) + bd0_ref[...])
    h = _elu(jnp.dot(h, wd2_ref[...], preferred_element_type=f32) + bd2_ref[...])
    out_ref[...] = jnp.dot(h, wd3_ref[...], preferred_element_type=f32) + bd3_ref[...]


def _tc_post(s_parts, d_parts, xpad, We3, be3, Wn0a, Wn0b, bn0,
             Wn2, bn2, Wn3, bn3, Wd0, bd0, Wd2, bd2, Wd3, bd3):
    grid = NPAD // _PRE_BLK
    row_blk = lambda i: (i, 0)
    full = lambda i: (0, 0)
    wspec = pl.BlockSpec((H, H), full)
    bspec = pl.BlockSpec((1, H), full)
    return pl.pallas_call(
        _tc_post_body,
        grid=(grid,),
        in_specs=(
            [pl.BlockSpec((1, _PRE_BLK, H), (lambda i, c=j % NC: (c, i, 0)))
             for j in range(NPART)]
            + [pl.BlockSpec((1, _PRE_BLK, 1), (lambda i, c=j % NC: (c, i, 0)))
               for j in range(NPART)]
            + [pl.BlockSpec((_PRE_BLK, H), row_blk),
               wspec, bspec, wspec, wspec, bspec,
               wspec, bspec, wspec, bspec,
               wspec, bspec, wspec, bspec, wspec, bspec]
        ),
        out_specs=pl.BlockSpec((_PRE_BLK, H), row_blk),
        out_shape=jax.ShapeDtypeStruct((NPAD, H), jnp.float32),
    )(*s_parts, *d_parts, xpad, We3, be3, Wn0a, Wn0b, bn0,
      Wn2, bn2, Wn3, bn3, Wd0, bd0, Wd2, bd2, Wd3, bd3)


def kernel(x, edge_index, pos, We0, be0, We2, be2, We3, be3,
           Wn0, bn0, Wn2, bn2, Wn3, bn3, Wd0, bd0, Wd2, bd2, Wd3, bd3):
    f32 = jnp.float32
    src = edge_index[0]
    dst = edge_index[1]
    E = src.shape[0]

    Wg = We0[:DIM]
    Wi = We0[DIM:DIM + D]
    Wj = We0[DIM + D:]
    zfill = jnp.zeros((256 - D - DIM, H), f32)
    wsrc = jnp.concatenate([Wi, -Wg, zfill], axis=0)
    wdst = jnp.concatenate([Wj, Wg, zfill], axis=0)

    xc = jnp.concatenate([x, pos], axis=1)
    xc = jnp.pad(xc, ((0, NPAD - N), (0, 256 - (D + DIM))))

    gsrc, gdst = _tc_pre(xc, wsrc, wdst)

    epad = EPAD - E
    spread = jnp.arange(epad, dtype=jnp.int32) % N
    srcg = jnp.concatenate([src, spread])
    dstg = jnp.concatenate([dst, spread])
    srcs = jnp.pad(src, (0, epad), constant_values=TRASH)
    eidx = jnp.stack(
        [srcg.reshape(-1, CH), dstg.reshape(-1, CH)], axis=1)

    csl = ESL // CH
    h0pre_list = []
    h1_list = []
    tok = jnp.zeros((1,), jnp.int32)
    for s in range(NSLICE):
        eidx_s = eidx[s * csl:(s + 1) * csl] + tok
        h0pre_s = _sc_gather(gsrc, gdst, eidx_s)
        h1_s = _tc_mid(h0pre_s, be0.reshape(1, H), We2, be2.reshape(1, H))
        tok = (h1_s[0, :1] * 0.0).astype(jnp.int32)
        h0pre_list.append(h0pre_s)
        h1_list.append(h1_s)

    s_parts = []
    d_parts = []
    for s in range(NSLICE):
        srcs_s = srcs[s * ESL:(s + 1) * ESL]
        if s + 1 < NSLICE:
            srcs_s = srcs_s + (h0pre_list[s + 1][0, :1] * 0.0).astype(jnp.int32)
        s_all, deg_all = _sc_scatter(h1_list[s], srcs_s)
        s3 = s_all.reshape(NC, NPAD, H)
        d3 = deg_all.reshape(NC, NPAD, 1)
        for c in range(NC):
            s_parts.append(s3)
            d_parts.append(d3)

    xpad = jnp.pad(x, ((0, NPAD - N), (0, 0)))

    out_pad = _tc_post(s_parts, d_parts, xpad,
                       We3, be3.reshape(1, H), Wn0[:D], Wn0[D:],
                       bn0.reshape(1, H), Wn2, bn2.reshape(1, H),
                       Wn3, bn3.reshape(1, H), Wd0, bd0.reshape(1, H),
                       Wd2, bd2.reshape(1, H), Wd3, bd3.reshape(1, H))
    return out_pad[:N]
